# baseline (speedup 1.0000x reference)
import functools

import jax
import jax.numpy as jnp
from jax import lax
from jax.experimental import pallas as pl
from jax.experimental.pallas import tpu as pltpu
from jax.experimental.pallas import tpu_sc as plsc

B = 16384
D = 16

_INFO = plsc.get_sparse_core_info()
_NC = _INFO.num_cores
_NS = _INFO.num_subcores
_NW = _NC * _NS
_BPW = B // _NW


def _gather_body(contrib_table, recip_table, contrib_idx, recip_idx,
                 xr_out, xc_out,
                 idx_r, idx_c, rows_r, rows_c,
                 sem_r, sem_c, sem_wr, sem_wc):
    wid = lax.axis_index("s") * _NC + lax.axis_index("c")
    base = wid * _BPW
    sl = pl.ds(base, _BPW)
    pltpu.sync_copy(recip_idx.at[sl], idx_r)
    pltpu.sync_copy(contrib_idx.at[sl], idx_c)
    gr = pltpu.async_copy(recip_table.at[idx_r], rows_r, sem_r)
    gc = pltpu.async_copy(contrib_table.at[idx_c], rows_c, sem_c)
    gr.wait()
    wr = pltpu.async_copy(rows_r, xr_out.at[sl], sem_wr)
    gc.wait()
    wc = pltpu.async_copy(rows_c, xc_out.at[sl], sem_wc)
    wr.wait()
    wc.wait()


@jax.jit
def kernel(contrib_table, recip_table, contrib_idx, recip_idx):
    mesh = plsc.VectorSubcoreMesh(core_axis_name="c", subcore_axis_name="s")
    out = pl.kernel(
        _gather_body,
        mesh=mesh,
        out_type=(
            jax.ShapeDtypeStruct((B, D), jnp.float32),
            jax.ShapeDtypeStruct((B, D), jnp.float32),
        ),
        scratch_types=[
            pltpu.VMEM((_BPW,), jnp.int32),
            pltpu.VMEM((_BPW,), jnp.int32),
            pltpu.VMEM((_BPW, D), jnp.float32),
            pltpu.VMEM((_BPW, D), jnp.float32),
            pltpu.SemaphoreType.DMA,
            pltpu.SemaphoreType.DMA,
            pltpu.SemaphoreType.DMA,
            pltpu.SemaphoreType.DMA,
        ],
        compiler_params=pltpu.CompilerParams(use_tc_tiling_on_sc=False),
    )(contrib_table, recip_table, contrib_idx, recip_idx)
    return out
